# baseline (speedup 1.0000x reference)
import functools

import jax
import jax.numpy as jnp
from jax import lax
from jax.experimental import pallas as pl
from jax.experimental.pallas import tpu as pltpu
from jax.experimental.pallas import tpu_sc as plsc

D_MODEL = 1024
D4 = D_MODEL // 4
NC, NS, L = 2, 16, 16
NW = NC * NS


def _sc_body(chunk, batch, minute_hbm, hour_hbm, day_hbm, month_hbm, out_hbm,
             idx_m, idx_h, idx_d, idx_mo, rows, sem):
    wid = lax.axis_index("s") * NC + lax.axis_index("c")
    base = wid * chunk

    for i in range(chunk // L):
        pos0 = jax.lax.broadcast_in_dim(base + i * L, (L,), ())
        pos = pos0 + lax.broadcasted_iota(jnp.int32, (L,), 0)
        sl = pl.ds(i * L, L)
        def splat(c):
            return jax.lax.broadcast_in_dim(jnp.int32(c), (L,), ())

        idx_m[sl] = lax.rem(pos, splat(60))
        idx_h[sl] = lax.rem(lax.div(pos, splat(60)), splat(24))
        idx_d[sl] = lax.rem(lax.div(pos, splat(60 * 24)), splat(32))
        idx_mo[sl] = lax.rem(lax.div(pos, splat(60 * 24 * 32)), splat(13))

    tables = ((minute_hbm, idx_m), (hour_hbm, idx_h),
              (day_hbm, idx_d), (month_hbm, idx_mo))
    for t, (tbl, idx) in enumerate(tables):
        pltpu.async_copy(tbl.at[idx], rows, sem).wait()
        for b in range(batch):
            pltpu.sync_copy(
                rows, out_hbm.at[b, pl.ds(base, chunk), pl.ds(t * D4, D4)])


def kernel(x, minute_table, hour_table, day_table, month_table):
    batch, seq_len, _ = x.shape
    chunk = seq_len // NW
    mesh = plsc.VectorSubcoreMesh(core_axis_name="c", subcore_axis_name="s",
                                  num_cores=NC, num_subcores=NS)

    run = pl.kernel(
        functools.partial(_sc_body, chunk, batch),
        out_type=jax.ShapeDtypeStruct((batch, seq_len, D_MODEL), jnp.float32),
        mesh=mesh,
        scratch_types=[
            pltpu.VMEM((chunk,), jnp.int32),
            pltpu.VMEM((chunk,), jnp.int32),
            pltpu.VMEM((chunk,), jnp.int32),
            pltpu.VMEM((chunk,), jnp.int32),
            pltpu.VMEM((chunk, D4), jnp.float32),
            pltpu.SemaphoreType.DMA,
        ],
    )
    return run(minute_table, hour_table, day_table, month_table)
